# Initial kernel scaffold; baseline (speedup 1.0000x reference)
import jax
import jax.numpy as jnp
from jax.experimental import pallas as pl


def kernel(x, params):
    raise NotImplementedError("write your pallas kernel here")


# speedup vs baseline: 1.3267x; 1.3267x over previous
import functools
from functools import partial

import jax
import jax.numpy as jnp
from jax import lax
from jax.experimental import pallas as pl
from jax.experimental.pallas import tpu as pltpu

F32 = jnp.float32
_PREC = None


def _fps_body(x_ref, out_ref, *, npoint, rows):
    xs = x_ref[0, 0]
    ys = x_ref[0, 1]
    zs = x_ref[0, 2]
    row = lax.broadcasted_iota(jnp.int32, (rows, 128), 0)
    col = lax.broadcasted_iota(jnp.int32, (rows, 128), 1)
    flat = row * 128 + col
    big = jnp.int32(rows * 128)

    def body(i, carry):
        fidx, dist = carry
        sel = flat == fidx
        zf = jnp.float32(0.0)
        cx = jnp.sum(jnp.where(sel, xs, zf))
        cy = jnp.sum(jnp.where(sel, ys, zf))
        cz = jnp.sum(jnp.where(sel, zs, zf))
        out_ref[0, pl.ds(i, 1), :] = jnp.stack([cx, cy, cz]).reshape(1, 3)
        dx = xs - cx
        dy = ys - cy
        dz = zs - cz
        d = (dx * dx + dy * dy) + dz * dz
        dist = jnp.minimum(dist, d)
        m = jnp.max(dist)
        fidx = jnp.min(jnp.where(dist == m, flat, big))
        return fidx, dist

    dist0 = jnp.full((rows, 128), 1e10, F32)
    lax.fori_loop(0, npoint, body, (jnp.int32(0), dist0))


def _fps(xyz, npoint):
    B, N, _ = xyz.shape
    rows = N // 128
    xt = jnp.transpose(xyz, (0, 2, 1)).reshape(B, 3, rows, 128)
    return pl.pallas_call(
        partial(_fps_body, npoint=npoint, rows=rows),
        grid=(B,),
        in_specs=[pl.BlockSpec((1, 3, rows, 128), lambda b: (b, 0, 0, 0))],
        out_specs=pl.BlockSpec((1, npoint, 3), lambda b: (b, 0, 0)),
        out_shape=jax.ShapeDtypeStruct((B, npoint, 3), F32),
    )(xt)


def _sqdist(src, dst):
    return (jnp.sum(src ** 2, -1)[..., :, None]
            + jnp.sum(dst ** 2, -1)[..., None, :]
            - 2.0 * jnp.matmul(src, jnp.swapaxes(dst, -1, -2)))


def _ball_idx(radius, nsample, xyz, new_xyz):
    B, S, _ = new_xyz.shape
    N = xyz.shape[1]
    d2 = _sqdist(new_xyz, xyz)
    gi = jnp.broadcast_to(jnp.arange(N, dtype=jnp.int32), (B, S, N))
    gi = jnp.where(d2 > radius ** 2, jnp.int32(N), gi)
    gi = jnp.sort(gi, axis=-1)[:, :, :nsample]
    first = gi[:, :, :1]
    return jnp.where(gi == N, first, gi)


def _pick_tq(M, per_row_elems, budget=262144):
    tq = max(1, budget // per_row_elems)
    if tq >= M:
        return M
    tq = max(8, tq - tq % 8)
    while tq > 0 and M % tq:
        tq -= 8
    return tq if tq > 0 else M


def _maxpool_body(x_ref, o_ref):
    o_ref[...] = jnp.max(x_ref[...], axis=1)


def _maxpool(x):
    M, K, C = x.shape
    tq = _pick_tq(M, K * C)
    return pl.pallas_call(
        _maxpool_body,
        grid=(M // tq,),
        in_specs=[pl.BlockSpec((tq, K, C), lambda i: (i, 0, 0))],
        out_specs=pl.BlockSpec((tq, C), lambda i: (i, 0)),
        out_shape=jax.ShapeDtypeStruct((M, C), F32),
    )(x)


def _fc_body(x_ref, w_ref, b_ref, o_ref):
    z = lax.dot_general(x_ref[...], w_ref[...], (((1,), (0,)), ((), ())),
                        preferred_element_type=F32, precision=_PREC)
    o_ref[...] = z + b_ref[...]


def _fc(x, w, b):
    M, Cin = x.shape
    Cout = w.shape[1]
    return pl.pallas_call(
        _fc_body,
        grid=(1,),
        in_specs=[pl.BlockSpec((M, Cin), lambda i: (0, 0)),
                  pl.BlockSpec((Cin, Cout), lambda i: (0, 0)),
                  pl.BlockSpec((1, Cout), lambda i: (0, 0))],
        out_specs=pl.BlockSpec((M, Cout), lambda i: (0, 0)),
        out_shape=jax.ShapeDtypeStruct((M, Cout), F32),
    )(x, w, b.reshape(1, Cout))


def _bn(x, gamma, beta, eps=1e-5):
    axes = tuple(range(x.ndim - 1))
    mean = jnp.mean(x, axis=axes, keepdims=True)
    var = jnp.var(x, axis=axes, keepdims=True)
    return gamma * (x - mean) / jnp.sqrt(var + eps) + beta


def _mlp(pts, layers):
    for (W, b, g, be) in layers:
        pts = jnp.matmul(pts, W) + b
        pts = jax.nn.relu(_bn(pts, g, be))
    return pts


def kernel(x, params):
    B, N, _ = x.shape
    xyz = x[:, :, :3]
    feats = x[:, :, 3:]
    batch = jnp.arange(B).reshape(B, 1, 1)

    nx1 = _fps(xyz, 2048)
    idx1 = _ball_idx(0.2, 64, xyz, nx1)
    grouped = jnp.concatenate(
        [xyz[batch, idx1] - nx1[:, :, None, :], feats[batch, idx1]], -1)
    a = _mlp(grouped, params['sa1'])
    l1 = _maxpool(a.reshape(-1, 64, 128)).reshape(B, 2048, 128)

    nx2 = _fps(nx1, 512)
    idx2 = _ball_idx(0.4, 128, nx1, nx2)
    grouped = jnp.concatenate(
        [nx1[batch, idx2] - nx2[:, :, None, :], l1[batch, idx2]], -1)
    a = _mlp(grouped, params['sa2'])
    l2 = _maxpool(a.reshape(-1, 128, 256)).reshape(B, 512, 256)

    grouped = jnp.concatenate([nx2, l2], axis=-1)[:, None, :, :]
    a = _mlp(grouped, params['sa3'])
    g = _maxpool(a.reshape(B, 512, 1024))

    (w1, b1) = params['fc1']
    (g1, be1) = params['bn1']
    h = jax.nn.relu(_bn(jnp.matmul(g, w1) + b1, g1, be1))
    (w2, b2) = params['fc2']
    (g2, be2) = params['bn2']
    h = jax.nn.relu(_bn(jnp.matmul(h, w2) + b2, g2, be2))
    (w3, b3) = params['fc3']
    return _fc(h, w3, b3)
